# speedup vs baseline: 5.4571x; 5.4571x over previous
import math

import jax
import jax.numpy as jnp
import numpy as np
from jax import lax
from jax.experimental import pallas as pl

N = 50000
E = 800000
NUM_TYPES = 4
N_BASIS = 8
LATENT = 128
MUL = 8
SH_DIM = 9
R_MAX = 5.0
POLY_P = 6.0
AVG_NEIGH = 16.0
SILU_CST = 1.6790

BE = 2000
BN = 2000

_PATH = np.array([0, 1, 1, 1, 2, 2, 2, 2, 2], dtype=np.int32)

_P1 = np.zeros((3 * MUL, MUL * SH_DIM), dtype=np.float32)
_P2 = np.zeros((SH_DIM, MUL * SH_DIM), dtype=np.float32)
for _m in range(MUL):
    for _l in range(SH_DIM):
        _P1[int(_PATH[_l]) * MUL + _m, _m * SH_DIM + _l] = 1.0
        _P2[_l, _m * SH_DIM + _l] = 1.0


def _edge_mlp_body(tt_ref, el_ref, sh_ref, bw_ref, w0p_ref, w0b_ref, w1_ref,
                   w2_ref, envw_ref, p1_ref, p2_ref, lat_ref, ef_ref):
    tt = tt_ref[0, 0, :]
    el = el_ref[0, 0, :]
    b = tt.shape[0]
    oh16 = (tt[:, None] == lax.broadcasted_iota(jnp.int32, (b, NUM_TYPES * NUM_TYPES), 1)).astype(jnp.float32)
    x = el[:, None]
    pref = math.sqrt(2.0 / R_MAX)
    basis = pref * jnp.sin(bw_ref[...] * x * (1.0 / R_MAX)) / x
    r = el * (1.0 / R_MAX)
    p = POLY_P
    cut = (1.0 - ((p + 1.0) * (p + 2.0) / 2.0) * r ** 6
           + p * (p + 2.0) * r ** 7
           - (p * (p + 1.0) / 2.0) * r ** 8)
    cut = jnp.where(el < R_MAX, cut, 0.0)
    h0 = jnp.dot(oh16, w0p_ref[...], preferred_element_type=jnp.float32)
    h0 = h0 + jnp.dot(basis, w0b_ref[...], preferred_element_type=jnp.float32)
    h0 = h0 * jax.nn.sigmoid(h0)
    h1 = jnp.dot(h0, w1_ref[...], preferred_element_type=jnp.float32)
    h1 = h1 * jax.nn.sigmoid(h1)
    lat = cut[:, None] * jnp.dot(h1, w2_ref[...], preferred_element_type=jnp.float32)
    lat_ref[...] = lat
    ew = jnp.dot(lat, envw_ref[...], preferred_element_type=jnp.float32) * cut[:, None]
    ef = (jnp.dot(ew, p1_ref[...], preferred_element_type=jnp.float32)
          * jnp.dot(sh_ref[...], p2_ref[...], preferred_element_type=jnp.float32))
    ef_ref[...] = ef


def _ln_body(nf_ref, ms_ref, srow_ref, wcol_ref, gamma_ref, bias_ref, out_ref):
    nf = nf_ref[...] * (1.0 / math.sqrt(AVG_NEIGH))
    smean = jnp.dot(nf, ms_ref[...], preferred_element_type=jnp.float32)
    nfc = nf - jnp.dot(smean, srow_ref[...], preferred_element_type=jnp.float32)
    sq = jnp.dot(nfc * nfc, wcol_ref[...], preferred_element_type=jnp.float32)
    inv = lax.rsqrt(sq + 1e-5)
    out_ref[...] = nfc * inv * gamma_ref[...] + bias_ref[...]


def _full(shape):
    return pl.BlockSpec(shape, lambda i: tuple(0 for _ in shape))


def kernel(edge_index, atom_type, bond_type, edge_sh, edge_length, node_one_hot,
           bessel_w, w0, w1, w2, env_w, ln_weight, ln_bias):
    center = edge_index[0]
    neigh = edge_index[1]
    tt = atom_type[center] * NUM_TYPES + atom_type[neigh]

    w0s = w0 * (1.0 / math.sqrt(float(w0.shape[0])))
    w0p = w0s[:NUM_TYPES][:, None, :] + w0s[NUM_TYPES:2 * NUM_TYPES][None, :, :]
    w0p = w0p.reshape(NUM_TYPES * NUM_TYPES, LATENT)
    w0b = w0s[2 * NUM_TYPES:]
    w1s = w1 * (SILU_CST / math.sqrt(float(LATENT)))
    w2s = w2 * (SILU_CST / math.sqrt(float(LATENT)))
    envs = env_w * (1.0 / math.sqrt(float(LATENT)))

    g = E // BE
    tt3 = tt.reshape(g, 1, BE).astype(jnp.int32)
    el3 = edge_length.reshape(g, 1, BE)

    latents, ef = pl.pallas_call(
        _edge_mlp_body,
        grid=(g,),
        in_specs=[
            pl.BlockSpec((1, 1, BE), lambda i: (i, 0, 0)),
            pl.BlockSpec((1, 1, BE), lambda i: (i, 0, 0)),
            pl.BlockSpec((BE, SH_DIM), lambda i: (i, 0)),
            _full((1, N_BASIS)),
            _full((NUM_TYPES * NUM_TYPES, LATENT)),
            _full((N_BASIS, LATENT)),
            _full((LATENT, LATENT)),
            _full((LATENT, LATENT)),
            _full((LATENT, 3 * MUL)),
            _full((3 * MUL, MUL * SH_DIM)),
            _full((SH_DIM, MUL * SH_DIM)),
        ],
        out_specs=[
            pl.BlockSpec((BE, LATENT), lambda i: (i, 0)),
            pl.BlockSpec((BE, MUL * SH_DIM), lambda i: (i, 0)),
        ],
        out_shape=[
            jax.ShapeDtypeStruct((E, LATENT), jnp.float32),
            jax.ShapeDtypeStruct((E, MUL * SH_DIM), jnp.float32),
        ],
    )(tt3, el3, edge_sh, bessel_w.reshape(1, N_BASIS), w0p, w0b, w1s, w2s,
      envs, jnp.asarray(_P1), jnp.asarray(_P2))

    nf = jax.ops.segment_sum(ef, center, num_segments=N)

    path = jnp.asarray(_PATH)
    is_s = (path == 0).astype(jnp.float32)
    ms = jnp.tile(is_s, MUL).reshape(MUL * SH_DIM, 1) * (1.0 / MUL)
    srow = jnp.tile(is_s, MUL).reshape(1, MUL * SH_DIM)
    wpath = jnp.asarray(np.array([1.0 / 24.0, 1.0 / 72.0, 1.0 / 120.0], np.float32))
    wcol = jnp.tile(wpath[path], MUL).reshape(MUL * SH_DIM, 1)
    gamma = jnp.transpose(ln_weight[path, :]).reshape(1, MUL * SH_DIM)
    bias = (jnp.tile(is_s, MUL) * jnp.repeat(ln_bias, SH_DIM)).reshape(1, MUL * SH_DIM)

    gn = N // BN
    node_out = pl.pallas_call(
        _ln_body,
        grid=(gn,),
        in_specs=[
            pl.BlockSpec((BN, MUL * SH_DIM), lambda i: (i, 0)),
            _full((MUL * SH_DIM, 1)),
            _full((1, MUL * SH_DIM)),
            _full((MUL * SH_DIM, 1)),
            _full((1, MUL * SH_DIM)),
            _full((1, MUL * SH_DIM)),
        ],
        out_specs=pl.BlockSpec((BN, MUL * SH_DIM), lambda i: (i, 0)),
        out_shape=jax.ShapeDtypeStruct((N, MUL * SH_DIM), jnp.float32),
    )(nf, ms, srow, wcol, gamma, bias)

    return latents, node_out.reshape(N, MUL, SH_DIM)


# speedup vs baseline: 17.9502x; 3.2893x over previous
import math

import jax
import jax.numpy as jnp
import numpy as np
from jax import lax
from jax.experimental import pallas as pl
from jax.experimental.pallas import tpu as pltpu
from jax.experimental.pallas import tpu_sc as plsc

N = 50000
E = 800000
NUM_TYPES = 4
N_BASIS = 8
LATENT = 128
MUL = 8
SH_DIM = 9
R_MAX = 5.0
POLY_P = 6.0
AVG_NEIGH = 16.0
SILU_CST = 1.6790

BE = 2000
BN = 2000

GATHER_WORKERS = 25
GCH = 2000

_PATH = np.array([0, 1, 1, 1, 2, 2, 2, 2, 2], dtype=np.int32)

_P1 = np.zeros((3 * MUL, MUL * SH_DIM), dtype=np.float32)
_P2 = np.zeros((SH_DIM, MUL * SH_DIM), dtype=np.float32)
for _m in range(MUL):
    for _l in range(SH_DIM):
        _c = _m * SH_DIM + _l
        _P1[int(_PATH[_l]) * MUL + _m, _c] = 1.0
        _P2[_l, _c] = 1.0


def _edge_mlp_body(tt_ref, el_ref, sh_ref, bw_ref, w0p_ref, w0b_ref, w1_ref,
                   w2_ref, envw_ref, p1_ref, p2_ref, lat_ref, ef_ref):
    tt = tt_ref[0, 0, :]
    el = el_ref[0, 0, :]
    b = tt.shape[0]
    oh16 = (tt[:, None] == lax.broadcasted_iota(
        jnp.int32, (b, NUM_TYPES * NUM_TYPES), 1)).astype(jnp.float32)
    x = el[:, None]
    pref = math.sqrt(2.0 / R_MAX)
    basis = pref * jnp.sin(bw_ref[...] * x * (1.0 / R_MAX)) / x
    r = el * (1.0 / R_MAX)
    p = POLY_P
    cut = (1.0 - ((p + 1.0) * (p + 2.0) / 2.0) * r ** 6
           + p * (p + 2.0) * r ** 7
           - (p * (p + 1.0) / 2.0) * r ** 8)
    cut = jnp.where(el < R_MAX, cut, 0.0)
    h0 = jnp.dot(oh16, w0p_ref[...], preferred_element_type=jnp.float32)
    h0 = h0 + jnp.dot(basis, w0b_ref[...], preferred_element_type=jnp.float32)
    h0 = h0 * jax.nn.sigmoid(h0)
    h1 = jnp.dot(h0, w1_ref[...], preferred_element_type=jnp.float32)
    h1 = h1 * jax.nn.sigmoid(h1)
    lat = cut[:, None] * jnp.dot(h1, w2_ref[...],
                                 preferred_element_type=jnp.float32)
    lat_ref[...] = lat
    ew = jnp.dot(lat, envw_ref[...],
                 preferred_element_type=jnp.float32) * cut[:, None]
    ef_ref[...] = (jnp.dot(ew, p1_ref[...], preferred_element_type=jnp.float32)
                   * jnp.dot(sh_ref[...], p2_ref[...],
                             preferred_element_type=jnp.float32))


def _ln_body(nf_ref, ms_ref, srow_ref, wcol_ref, gamma_ref, bias_ref, out_ref):
    nf = nf_ref[...] * (1.0 / math.sqrt(AVG_NEIGH))
    smean = jnp.dot(nf, ms_ref[...], preferred_element_type=jnp.float32)
    nfc = nf - jnp.dot(smean, srow_ref[...], preferred_element_type=jnp.float32)
    sq = jnp.dot(nfc * nfc, wcol_ref[...], preferred_element_type=jnp.float32)
    inv = lax.rsqrt(sq + 1e-5)
    out_ref[...] = nfc * inv * gamma_ref[...] + bias_ref[...]


def _full(shape):
    return pl.BlockSpec(shape, lambda i: tuple(0 for _ in shape))


def _tt_gather_body(at_hbm, c_hbm, n_hbm, tt_hbm, at_v, c_v, n_v, tt_v):
    cid = lax.axis_index("c")
    sid = lax.axis_index("s")
    w = sid * 2 + cid
    pltpu.sync_copy(at_hbm, at_v)

    @pl.when(w < GATHER_WORKERS)
    def _work():
        base = w * (E // GATHER_WORKERS)

        def chunk(i, carry):
            off = base + i * GCH
            pltpu.sync_copy(c_hbm.at[pl.ds(off, GCH)], c_v)
            pltpu.sync_copy(n_hbm.at[pl.ds(off, GCH)], n_v)

            def grp(j, carry2):
                cg = c_v[pl.ds(j * 16, 16)]
                ng = n_v[pl.ds(j * 16, 16)]
                ac = plsc.load_gather(at_v, [cg])
                an = plsc.load_gather(at_v, [ng])
                tt_v[pl.ds(j * 16, 16)] = ac * NUM_TYPES + an
                return carry2

            lax.fori_loop(0, GCH // 16, grp, 0)
            pltpu.sync_copy(tt_v, tt_hbm.at[pl.ds(off, GCH)])
            return carry

        lax.fori_loop(0, E // GATHER_WORKERS // GCH, chunk, 0)


_SC_MESH = plsc.VectorSubcoreMesh(core_axis_name="c", subcore_axis_name="s")


def kernel(edge_index, atom_type, bond_type, edge_sh, edge_length, node_one_hot,
           bessel_w, w0, w1, w2, env_w, ln_weight, ln_bias):
    center = edge_index[0].astype(jnp.int32)
    neigh = edge_index[1].astype(jnp.int32)

    tt = pl.kernel(
        _tt_gather_body,
        out_type=jax.ShapeDtypeStruct((E,), jnp.int32),
        mesh=_SC_MESH,
        scratch_types=[
            pltpu.VMEM((N,), jnp.int32),
            pltpu.VMEM((GCH,), jnp.int32),
            pltpu.VMEM((GCH,), jnp.int32),
            pltpu.VMEM((GCH,), jnp.int32),
        ],
        compiler_params=pltpu.CompilerParams(needs_layout_passes=False),
    )(atom_type.astype(jnp.int32), center, neigh)

    w0s = w0 * (1.0 / math.sqrt(float(w0.shape[0])))
    w0p = w0s[:NUM_TYPES][:, None, :] + w0s[NUM_TYPES:2 * NUM_TYPES][None, :, :]
    w0p = w0p.reshape(NUM_TYPES * NUM_TYPES, LATENT)
    w0b = w0s[2 * NUM_TYPES:]
    w1s = w1 * (SILU_CST / math.sqrt(float(LATENT)))
    w2s = w2 * (SILU_CST / math.sqrt(float(LATENT)))
    envs = env_w * (1.0 / math.sqrt(float(LATENT)))

    g = E // BE
    tt3 = tt.reshape(g, 1, BE)
    el3 = edge_length.reshape(g, 1, BE)

    latents, ef = pl.pallas_call(
        _edge_mlp_body,
        grid=(g,),
        in_specs=[
            pl.BlockSpec((1, 1, BE), lambda i: (i, 0, 0)),
            pl.BlockSpec((1, 1, BE), lambda i: (i, 0, 0)),
            pl.BlockSpec((BE, SH_DIM), lambda i: (i, 0)),
            _full((1, N_BASIS)),
            _full((NUM_TYPES * NUM_TYPES, LATENT)),
            _full((N_BASIS, LATENT)),
            _full((LATENT, LATENT)),
            _full((LATENT, LATENT)),
            _full((LATENT, 3 * MUL)),
            _full((3 * MUL, MUL * SH_DIM)),
            _full((SH_DIM, MUL * SH_DIM)),
        ],
        out_specs=[
            pl.BlockSpec((BE, LATENT), lambda i: (i, 0)),
            pl.BlockSpec((BE, MUL * SH_DIM), lambda i: (i, 0)),
        ],
        out_shape=[
            jax.ShapeDtypeStruct((E, LATENT), jnp.float32),
            jax.ShapeDtypeStruct((E, MUL * SH_DIM), jnp.float32),
        ],
    )(tt3, el3, edge_sh, bessel_w.reshape(1, N_BASIS), w0p, w0b, w1s, w2s,
      envs, jnp.asarray(_P1), jnp.asarray(_P2))

    nf = jax.ops.segment_sum(ef, center, num_segments=N)

    path = jnp.asarray(_PATH)
    is_s = (path == 0).astype(jnp.float32)
    ms = jnp.tile(is_s, MUL).reshape(MUL * SH_DIM, 1) * (1.0 / MUL)
    srow = jnp.tile(is_s, MUL).reshape(1, MUL * SH_DIM)
    wpath = jnp.asarray(np.array([1.0 / 24.0, 1.0 / 72.0, 1.0 / 120.0],
                                 np.float32))
    wcol = jnp.tile(wpath[path], MUL).reshape(MUL * SH_DIM, 1)
    gamma = jnp.transpose(ln_weight[path, :]).reshape(1, MUL * SH_DIM)
    bias = (jnp.tile(is_s, MUL) * jnp.repeat(ln_bias, SH_DIM)).reshape(
        1, MUL * SH_DIM)

    gn = N // BN
    node_out = pl.pallas_call(
        _ln_body,
        grid=(gn,),
        in_specs=[
            pl.BlockSpec((BN, MUL * SH_DIM), lambda i: (i, 0)),
            _full((MUL * SH_DIM, 1)),
            _full((1, MUL * SH_DIM)),
            _full((MUL * SH_DIM, 1)),
            _full((1, MUL * SH_DIM)),
            _full((1, MUL * SH_DIM)),
        ],
        out_specs=pl.BlockSpec((BN, MUL * SH_DIM), lambda i: (i, 0)),
        out_shape=jax.ShapeDtypeStruct((N, MUL * SH_DIM), jnp.float32),
    )(nf, ms, srow, wcol, gamma, bias)

    return latents, node_out.reshape(N, MUL, SH_DIM)
